# baseline (speedup 1.0000x reference)
import functools

import jax
import jax.numpy as jnp
from jax import lax
from jax.experimental import pallas as pl
from jax.experimental.pallas import tpu as pltpu
from jax.experimental.pallas import tpu_sc as plsc

TM = 1024
KC = 1024


def _vq_tc_body(x_ref, k_ref, acc_in_ref, idx_ref, kpad_ref, scal_ref,
                kT2_ref, ksq_ref, dall_ref, acc_ref, *,
                n_code, n_steps, w, emit_kpad, finish, total_tok):
    pid = pl.program_id(0)

    @pl.when(pid == 0)
    def _prep():
        kk = k_ref[...]
        if emit_kpad:
            kpad_ref[:, :w] = kk
            kpad_ref[:, w:] = jnp.zeros_like(kpad_ref[:, w:])
        kT2 = jnp.transpose(kk, (1, 0)) * jnp.float32(-2.0)
        kT2_ref[...] = kT2
        ksq_ref[...] = 0.25 * jnp.sum(kT2 * kT2, axis=0, keepdims=True)

    xb = jnp.transpose(x_ref[0], (1, 0))
    xsq = jnp.sum(xb * xb, axis=1)
    cmin = None
    for c in range(n_code // KC):
        kTc = kT2_ref[:, c * KC:(c + 1) * KC]
        s2 = lax.dot_general(xb, kTc, (((1,), (0,)), ((), ())),
                             preferred_element_type=jnp.float32)
        d = (xsq[:, None] + s2) + ksq_ref[:, c * KC:(c + 1) * KC]
        dall_ref[:, c * KC:(c + 1) * KC] = d
        cm = jnp.min(d, axis=1, keepdims=True)
        cmin = cm if cmin is None else jnp.minimum(cmin, cm)
    io = lax.broadcasted_iota(jnp.int32, (1, n_code), 1).astype(jnp.float32)
    cand = jnp.where(dall_ref[...] == cmin,
                     jnp.broadcast_to(io, (TM, n_code)), jnp.float32(2 ** 30))
    idx_ref[0, 0, :] = jnp.min(cand, axis=1).astype(jnp.int32)

    lanes = lax.broadcasted_iota(jnp.int32, (1, 128), 1)
    row = (jnp.where(lanes == 0, jnp.sum(cmin), 0.0)
           + jnp.where(lanes == 1, jnp.sum(xb), 0.0)
           + jnp.where(lanes == 2, jnp.sum(xsq), 0.0))

    @pl.when(pid == 0)
    def _init_acc():
        acc_ref[...] = acc_in_ref[...] + row

    @pl.when(pid > 0)
    def _add_acc():
        acc_ref[...] = acc_ref[...] + row

    @pl.when(pid == n_steps - 1)
    def _finish():
        if finish:
            n_tok = jnp.float32(total_tok)
            n_el = n_tok * w
            sm = acc_ref[0, 0]
            s1 = acc_ref[0, 1]
            s2v = acc_ref[0, 2]
            commit = sm / n_el
            fit = sm / n_tok
            pre = jnp.sqrt((s2v - s1 * s1 / n_el) / n_el)
            scal_ref[...] = (jnp.where(lanes == 0, commit, 0.0)
                             + jnp.where(lanes == 1, fit, 0.0)
                             + jnp.where(lanes == 2, pre, 0.0))
        else:
            scal_ref[...] = acc_ref[...]


def _quantise(x, k, acc_in, n_offset, n_batches, emit_kpad, finish,
              total_tok):
    _, w, t = x.shape
    n_tok = n_batches * t
    n_code = k.shape[0]
    n_steps = n_tok // TM
    bpn = t // TM
    grid = (n_steps,)

    kw = dict(n_code=n_code, n_steps=n_steps, w=w, emit_kpad=emit_kpad,
              finish=finish, total_tok=total_tok)
    if emit_kpad:
        body = functools.partial(_vq_tc_body, **kw)
    else:
        def body(x_ref, k_ref, acc_in_ref, idx_ref, scal_ref, *scr):
            _vq_tc_body(x_ref, k_ref, acc_in_ref, idx_ref, None, scal_ref,
                        *scr, **kw)

    out_specs = [pl.BlockSpec((1, 1, TM), lambda i: (i, 0, 0))]
    out_shape = [jax.ShapeDtypeStruct((n_steps, 1, TM), jnp.int32)]
    if emit_kpad:
        out_specs.append(pl.BlockSpec((n_code, 128), lambda i: (0, 0)))
        out_shape.append(jax.ShapeDtypeStruct((n_code, 128), jnp.float32))
    out_specs.append(pl.BlockSpec((1, 128), lambda i: (0, 0)))
    out_shape.append(jax.ShapeDtypeStruct((1, 128), jnp.float32))

    outs = pl.pallas_call(
        body,
        grid=grid,
        in_specs=[
            pl.BlockSpec((1, w, TM),
                         lambda i, o=n_offset, b=bpn: (o + i // b, 0, i % b)),
            pl.BlockSpec((n_code, w), lambda i: (0, 0)),
            pl.BlockSpec((1, 128), lambda i: (0, 0)),
        ],
        out_specs=out_specs,
        out_shape=out_shape,
        scratch_shapes=[pltpu.VMEM((w, n_code), jnp.float32),
                        pltpu.VMEM((1, n_code), jnp.float32),
                        pltpu.VMEM((TM, n_code), jnp.float32),
                        pltpu.VMEM((1, 128), jnp.float32)],
    )(x, k, acc_in)
    if emit_kpad:
        idx3, kpad, scal = outs
        return idx3.reshape(-1), kpad, scal
    idx3, scal = outs
    return idx3.reshape(-1), None, scal


def _make_sc_gather(n_tok, w):
    info = plsc.get_sparse_core_info()
    nw = info.num_cores * info.num_subcores
    b_per_w = n_tok // nw
    chunk = 128
    n_chunks = b_per_w // chunk
    mesh = plsc.VectorSubcoreMesh(core_axis_name="c", subcore_axis_name="s")

    @functools.partial(
        pl.kernel, mesh=mesh,
        out_type=jax.ShapeDtypeStruct((n_tok, w), jnp.float32),
        scratch_types=[
            pltpu.VMEM((b_per_w,), jnp.int32),
            pltpu.VMEM((b_per_w, w), jnp.float32),
            pltpu.SemaphoreType.DMA,
        ],
    )
    def gather_rows(k_hbm, idx_hbm, out_hbm, idx_v, rows_v, sem):
        wid = lax.axis_index("s") * info.num_cores + lax.axis_index("c")
        base = wid * b_per_w
        pltpu.sync_copy(idx_hbm.at[pl.ds(base, b_per_w)], idx_v)
        copies = []
        for j in range(n_chunks):
            copies.append(pltpu.async_copy(
                k_hbm.at[idx_v.at[pl.ds(j * chunk, chunk)]],
                rows_v.at[pl.ds(j * chunk, chunk), :], sem))
        for cp in copies:
            cp.wait()
        pltpu.sync_copy(rows_v, out_hbm.at[pl.ds(base, b_per_w)])

    return gather_rows


def kernel(x, k):
    n, width, t = x.shape
    nh = n // 2
    zrow = jnp.zeros((1, 128), jnp.float32)

    xl1, kpad, acc1 = _quantise(x, k, zrow, 0, nh, True, False, n * t)
    xd1 = _make_sc_gather(nh * t, 128)(kpad, xl1)[:, :width]
    xl2, _, scal = _quantise(x, k, acc1, nh, nh, False, True, n * t)
    xd2 = _make_sc_gather(nh * t, 128)(kpad, xl2)[:, :width]

    commit_loss = scal[0, 0]
    fit = scal[0, 1]
    prenorm = scal[0, 2]

    x_l_out = jnp.concatenate([xl1.reshape(nh, t), xl2.reshape(nh, t)], axis=0)
    x_d_out = jnp.concatenate(
        [jnp.transpose(xd1.reshape(nh, t, width), (0, 2, 1)),
         jnp.transpose(xd2.reshape(nh, t, width), (0, 2, 1))], axis=0)
    return (x_l_out, x_d_out, commit_loss, fit, prenorm)


# speedup vs baseline: 1.0117x; 1.0117x over previous
import functools

import jax
import jax.numpy as jnp
from jax import lax
from jax.experimental import pallas as pl
from jax.experimental.pallas import tpu as pltpu
from jax.experimental.pallas import tpu_sc as plsc

TM = 1024
KC = 2048


def _vq_tc_body(x_ref, k_ref, idx_ref, kpad_ref, scal_ref,
                kT2_ref, ksq_ref, dall_ref, acc_ref, *, n_code, n_steps, w):
    pid = pl.program_id(0)

    @pl.when(pid == 0)
    def _prep():
        kk = k_ref[...]
        kpad_ref[:, :w] = kk
        kpad_ref[:, w:] = jnp.zeros_like(kpad_ref[:, w:])
        kT2 = jnp.transpose(kk, (1, 0)) * jnp.float32(-2.0)
        kT2_ref[...] = kT2
        ksq_ref[...] = 0.25 * jnp.sum(kT2 * kT2, axis=0, keepdims=True)

    xb = jnp.transpose(x_ref[0], (1, 0))
    xsq = jnp.sum(xb * xb, axis=1)
    cmin = None
    for c in range(n_code // KC):
        kTc = kT2_ref[:, c * KC:(c + 1) * KC]
        s2 = lax.dot_general(xb, kTc, (((1,), (0,)), ((), ())),
                             preferred_element_type=jnp.float32)
        d = (xsq[:, None] + s2) + ksq_ref[:, c * KC:(c + 1) * KC]
        dall_ref[:, c * KC:(c + 1) * KC] = d
        cm = jnp.min(d, axis=1, keepdims=True)
        cmin = cm if cmin is None else jnp.minimum(cmin, cm)
    io = lax.broadcasted_iota(jnp.int32, (1, n_code), 1).astype(jnp.float32)
    cand = jnp.where(dall_ref[...] == cmin,
                     jnp.broadcast_to(io, (TM, n_code)), jnp.float32(2 ** 30))
    idx_ref[0, 0, :] = jnp.min(cand, axis=1).astype(jnp.int32)

    lanes = lax.broadcasted_iota(jnp.int32, (1, 128), 1)
    row = (jnp.where(lanes == 0, jnp.sum(cmin), 0.0)
           + jnp.where(lanes == 1, jnp.sum(xb), 0.0)
           + jnp.where(lanes == 2, jnp.sum(xsq), 0.0))

    @pl.when(pid == 0)
    def _init_acc():
        acc_ref[...] = row

    @pl.when(pid > 0)
    def _add_acc():
        acc_ref[...] = acc_ref[...] + row

    @pl.when(pid == n_steps - 1)
    def _finish():
        n_tok = jnp.float32(n_steps * TM)
        n_el = n_tok * w
        sm = acc_ref[0, 0]
        s1 = acc_ref[0, 1]
        s2v = acc_ref[0, 2]
        commit = sm / n_el
        fit = sm / n_tok
        pre = jnp.sqrt((s2v - s1 * s1 / n_el) / n_el)
        scal_ref[...] = (jnp.where(lanes == 0, commit, 0.0)
                         + jnp.where(lanes == 1, fit, 0.0)
                         + jnp.where(lanes == 2, pre, 0.0))


def _quantise(x, k):
    n, w, t = x.shape
    n_tok = n * t
    n_code = k.shape[0]
    n_steps = n_tok // TM
    blocks_per_n = t // TM
    grid = (n_steps,)
    idx3, kpad, scal = pl.pallas_call(
        functools.partial(_vq_tc_body, n_code=n_code, n_steps=n_steps, w=w),
        grid=grid,
        in_specs=[
            pl.BlockSpec((1, w, TM),
                         lambda i: (i // blocks_per_n, 0, i % blocks_per_n)),
            pl.BlockSpec((n_code, w), lambda i: (0, 0)),
        ],
        out_specs=[pl.BlockSpec((1, 1, TM), lambda i: (i, 0, 0)),
                   pl.BlockSpec((n_code, 128), lambda i: (0, 0)),
                   pl.BlockSpec((1, 128), lambda i: (0, 0))],
        out_shape=[jax.ShapeDtypeStruct((n_steps, 1, TM), jnp.int32),
                   jax.ShapeDtypeStruct((n_code, 128), jnp.float32),
                   jax.ShapeDtypeStruct((1, 128), jnp.float32)],
        scratch_shapes=[pltpu.VMEM((w, n_code), jnp.float32),
                        pltpu.VMEM((1, n_code), jnp.float32),
                        pltpu.VMEM((TM, n_code), jnp.float32),
                        pltpu.VMEM((1, 128), jnp.float32)],
    )(x, k)
    return idx3.reshape(-1), kpad, scal


def _make_sc_gather(n_tok, w):
    info = plsc.get_sparse_core_info()
    nw = info.num_cores * info.num_subcores
    b_per_w = n_tok // nw
    chunk = 128
    n_chunks = b_per_w // chunk
    mesh = plsc.VectorSubcoreMesh(core_axis_name="c", subcore_axis_name="s")

    @functools.partial(
        pl.kernel, mesh=mesh,
        out_type=jax.ShapeDtypeStruct((n_tok, w), jnp.float32),
        scratch_types=[
            pltpu.VMEM((b_per_w,), jnp.int32),
            pltpu.VMEM((b_per_w, w), jnp.float32),
            pltpu.SemaphoreType.DMA,
        ],
    )
    def gather_rows(k_hbm, idx_hbm, out_hbm, idx_v, rows_v, sem):
        wid = lax.axis_index("s") * info.num_cores + lax.axis_index("c")
        base = wid * b_per_w
        pltpu.sync_copy(idx_hbm.at[pl.ds(base, b_per_w)], idx_v)
        copies = []
        for j in range(n_chunks):
            copies.append(pltpu.async_copy(
                k_hbm.at[idx_v.at[pl.ds(j * chunk, chunk)]],
                rows_v.at[pl.ds(j * chunk, chunk), :], sem))
        for cp in copies:
            cp.wait()
        pltpu.sync_copy(rows_v, out_hbm.at[pl.ds(base, b_per_w)])

    return gather_rows


def kernel(x, k):
    n, width, t = x.shape
    n_tok = n * t

    x_l, kpad, scal = _quantise(x, k)
    x_d = _make_sc_gather(n_tok, 128)(kpad, x_l)[:, :width]

    commit_loss = scal[0, 0]
    fit = scal[0, 1]
    prenorm = scal[0, 2]

    x_l_out = x_l.reshape(n, t)
    x_d_out = jnp.transpose(x_d.reshape(n, t, width), (0, 2, 1))
    return (x_l_out, x_d_out, commit_loss, fit, prenorm)
